# baseline (speedup 1.0000x reference)
import functools

import jax
import jax.numpy as jnp
from jax import lax
from jax.experimental import pallas as pl
from jax.experimental.pallas import tpu as pltpu
from jax.experimental.pallas import tpu_sc as plsc

SEQ_LEN = 2048
IN_CH = 1024
OUT_CH = 4096
N_IMG = 576
PLACEHOLDER_POS = 32
OUT_ROWS = SEQ_LEN - 1 + N_IMG + 2
PROJ_ROWS = N_IMG + 2

NC, NS = 2, 16
NW = NC * NS
GCHUNK = 8
NGCHUNK = 257
IDX_LEN = NGCHUNK * GCHUNK
CHUNKS_PER_W = 8
PRE_CHUNKS = PLACEHOLDER_POS // GCHUNK
NBUF = 3

NJ = 8
BLK = OUT_CH // NJ


def _mlp_body(img_ref, w1_ref, b1_ref, w2_ref, b2_ref, start_ref, end_ref,
              out_ref, h_ref):
    @pl.when(pl.program_id(0) == 0)
    def _():
        h = jnp.dot(img_ref[...], w1_ref[...],
                    precision=lax.Precision.DEFAULT,
                    preferred_element_type=jnp.float32) + b1_ref[...]
        h_ref[...] = 0.5 * h * (1.0 + lax.erf(h * 0.7071067811865476))
    out_ref[0:1, :] = start_ref[...]
    out_ref[1:N_IMG + 1, :] = jnp.dot(h_ref[...], w2_ref[...],
                                      precision=lax.Precision.DEFAULT,
                                      preferred_element_type=jnp.float32
                                      ) + b2_ref[...]
    out_ref[N_IMG + 1:PROJ_ROWS, :] = end_ref[...]


def _mlp(img2d, W1, b1, W2, b2, start_row, end_row):
    return pl.pallas_call(
        _mlp_body,
        grid=(NJ,),
        in_specs=[
            pl.BlockSpec((N_IMG, IN_CH), lambda j: (0, 0)),
            pl.BlockSpec((IN_CH, OUT_CH), lambda j: (0, 0)),
            pl.BlockSpec((1, OUT_CH), lambda j: (0, 0)),
            pl.BlockSpec((OUT_CH, BLK), lambda j: (0, j)),
            pl.BlockSpec((1, BLK), lambda j: (0, j)),
            pl.BlockSpec((1, BLK), lambda j: (0, j)),
            pl.BlockSpec((1, BLK), lambda j: (0, j)),
        ],
        out_specs=pl.BlockSpec((PROJ_ROWS, BLK), lambda j: (0, j)),
        out_shape=jax.ShapeDtypeStruct((PROJ_ROWS, OUT_CH), jnp.float32),
        scratch_shapes=[pltpu.VMEM((N_IMG, OUT_CH), jnp.float32)],
    )(img2d, W1, b1.reshape(1, OUT_CH), W2, b2.reshape(1, OUT_CH),
      start_row, end_row)


@functools.lru_cache(maxsize=None)
def _make_gather():
    return pl.kernel(
        _gather_body,
        out_type=jax.ShapeDtypeStruct((OUT_ROWS, OUT_CH), jnp.float32),
        mesh=plsc.VectorSubcoreMesh(core_axis_name="c", subcore_axis_name="s",
                                    num_cores=NC, num_subcores=NS),
        scratch_types=[
            pltpu.VMEM((CHUNKS_PER_W * GCHUNK,), jnp.int32),
            pltpu.VMEM((NBUF, GCHUNK, OUT_CH), jnp.float32),
            pltpu.SemaphoreType.DMA,
        ],
    )


def _gather_body(idx_hbm, table_hbm, out_hbm, idx_v, rows_v, gsem):
    wid = lax.axis_index("s") * NC + lax.axis_index("c")
    base = wid * (CHUNKS_PER_W * GCHUNK)
    pltpu.sync_copy(idx_hbm.at[pl.ds(base, CHUNKS_PER_W * GCHUNK)], idx_v)
    gathers = []
    for c in range(CHUNKS_PER_W):
        gathers.append(pltpu.async_copy(
            table_hbm.at[idx_v.at[pl.ds(c * GCHUNK, GCHUNK)]],
            rows_v.at[c % NBUF], gsem))
        if c >= NBUF - 1:
            _chunk_store(wid, c - NBUF + 1, rows_v, gathers[c - NBUF + 1],
                         out_hbm)
    for c in range(CHUNKS_PER_W - NBUF + 1, CHUNKS_PER_W):
        _chunk_store(wid, c, rows_v, gathers[c], out_hbm)
    @pl.when(wid == NW - 1)
    def _():
        pltpu.sync_copy(idx_hbm.at[pl.ds(NW * CHUNKS_PER_W * GCHUNK, GCHUNK)],
                        idx_v.at[pl.ds(0, GCHUNK)])
        pltpu.async_copy(
            table_hbm.at[idx_v.at[pl.ds(0, GCHUNK)]], rows_v.at[0],
            gsem).wait()
        pltpu.sync_copy(rows_v.at[0, pl.ds(0, 1)],
                        out_hbm.at[pl.ds(OUT_ROWS - 1, 1)])


def _chunk_store(wid, c, rows_v, gather, out_hbm):
    cid = wid * CHUNKS_PER_W + c
    off = cid * GCHUNK + jnp.where(cid < PRE_CHUNKS, 0, PROJ_ROWS - 2)
    off = pl.multiple_of(off, GCHUNK)
    gather.wait()
    pltpu.sync_copy(rows_v.at[c % NBUF], out_hbm.at[pl.ds(off, GCHUNK)])


def _finish_body(proj_hbm, out0_hbm, out_hbm, buf, buf2, sem, sem2):
    cp_in = pltpu.make_async_copy(proj_hbm, buf, sem)
    cp_in.start()
    cp_bnd = pltpu.make_async_copy(
        out0_hbm.at[pl.ds(PLACEHOLDER_POS + N_IMG, 8), :], buf2, sem2)
    cp_bnd.start()
    cp_in.wait()
    cp_bnd.wait()
    buf2[0:2, :] = buf[N_IMG:PROJ_ROWS, :]
    cp_out = pltpu.make_async_copy(
        buf.at[pl.ds(0, N_IMG)],
        out_hbm.at[pl.ds(PLACEHOLDER_POS, N_IMG), :], sem)
    cp_out.start()
    cp_out2 = pltpu.make_async_copy(
        buf2, out_hbm.at[pl.ds(PLACEHOLDER_POS + N_IMG, 8), :], sem2)
    cp_out2.start()
    cp_out.wait()
    cp_out2.wait()


def _finish(proj_ext, out0):
    return pl.pallas_call(
        _finish_body,
        in_specs=[pl.BlockSpec(memory_space=pl.ANY),
                  pl.BlockSpec(memory_space=pl.ANY)],
        out_specs=pl.BlockSpec(memory_space=pl.ANY),
        out_shape=jax.ShapeDtypeStruct((OUT_ROWS, OUT_CH), jnp.float32),
        scratch_shapes=[pltpu.VMEM((PROJ_ROWS, OUT_CH), jnp.float32),
                        pltpu.VMEM((8, OUT_CH), jnp.float32),
                        pltpu.SemaphoreType.DMA,
                        pltpu.SemaphoreType.DMA],
        input_output_aliases={1: 0},
    )(proj_ext, out0)


def kernel(instruction, img_token, embed_table, W1, b1, W2, b2,
           visual_start_token, visual_end_token):
    zeros2 = jnp.zeros((2,), jnp.int32)
    zpad = jnp.zeros((IDX_LEN - SEQ_LEN - 1,), jnp.int32)
    idx_sc = jnp.concatenate([
        instruction[:PLACEHOLDER_POS], zeros2,
        instruction[PLACEHOLDER_POS + 1:SEQ_LEN - 1],
        instruction[SEQ_LEN - 1:], zpad,
    ])
    out0 = _make_gather()(idx_sc, embed_table)
    proj_ext = _mlp(img_token.reshape(N_IMG, IN_CH), W1, b1, W2, b2,
                    visual_start_token.reshape(1, OUT_CH),
                    visual_end_token.reshape(1, OUT_CH))
    out_lin = lax.optimization_barrier(out0[None])
    return lax.dynamic_update_slice(out_lin, proj_ext[None],
                                    (0, PLACEHOLDER_POS, 0))


# speedup vs baseline: 1.1657x; 1.1657x over previous
import functools

import jax
import jax.numpy as jnp
from jax import lax
from jax.experimental import pallas as pl
from jax.experimental.pallas import tpu as pltpu
from jax.experimental.pallas import tpu_sc as plsc

SEQ_LEN = 2048
IN_CH = 1024
OUT_CH = 4096
N_IMG = 576
PLACEHOLDER_POS = 32
OUT_ROWS = SEQ_LEN - 1 + N_IMG + 2
PROJ_ROWS = N_IMG + 2

NC, NS = 2, 16
NW = NC * NS
GCHUNK = 8
NGCHUNK = 257
IDX_LEN = NGCHUNK * GCHUNK
CHUNKS_PER_W = 8
PRE_CHUNKS = PLACEHOLDER_POS // GCHUNK
NBUF = 3

NJ = 8
BLK = OUT_CH // NJ


def _mlp_body(img_ref, w1_ref, b1_ref, w2_ref, b2_ref, start_ref, end_ref,
              out_ref, h_ref):
    @pl.when(pl.program_id(0) == 0)
    def _():
        h = jnp.dot(img_ref[...], w1_ref[...],
                    precision=lax.Precision.DEFAULT,
                    preferred_element_type=jnp.float32) + b1_ref[...]
        h_ref[...] = 0.5 * h * (1.0 + lax.erf(h * 0.7071067811865476))
    out_ref[0:1, :] = start_ref[...]
    out_ref[1:N_IMG + 1, :] = jnp.dot(h_ref[...], w2_ref[...],
                                      precision=lax.Precision.DEFAULT,
                                      preferred_element_type=jnp.float32
                                      ) + b2_ref[...]
    out_ref[N_IMG + 1:PROJ_ROWS, :] = end_ref[...]


def _mlp(img2d, W1, b1, W2, b2, start_row, end_row):
    return pl.pallas_call(
        _mlp_body,
        grid=(NJ,),
        in_specs=[
            pl.BlockSpec((N_IMG, IN_CH), lambda j: (0, 0)),
            pl.BlockSpec((IN_CH, OUT_CH), lambda j: (0, 0)),
            pl.BlockSpec((1, OUT_CH), lambda j: (0, 0)),
            pl.BlockSpec((OUT_CH, BLK), lambda j: (0, j)),
            pl.BlockSpec((1, BLK), lambda j: (0, j)),
            pl.BlockSpec((1, BLK), lambda j: (0, j)),
            pl.BlockSpec((1, BLK), lambda j: (0, j)),
        ],
        out_specs=pl.BlockSpec((PROJ_ROWS, BLK), lambda j: (0, j)),
        out_shape=jax.ShapeDtypeStruct((PROJ_ROWS, OUT_CH), jnp.float32),
        scratch_shapes=[pltpu.VMEM((N_IMG, OUT_CH), jnp.float32)],
    )(img2d, W1, b1.reshape(1, OUT_CH), W2, b2.reshape(1, OUT_CH),
      start_row, end_row)


@functools.lru_cache(maxsize=None)
def _make_gather():
    return pl.kernel(
        _gather_body,
        out_type=jax.ShapeDtypeStruct((OUT_ROWS, OUT_CH), jnp.float32),
        mesh=plsc.VectorSubcoreMesh(core_axis_name="c", subcore_axis_name="s",
                                    num_cores=NC, num_subcores=NS),
        scratch_types=[
            pltpu.VMEM((CHUNKS_PER_W * GCHUNK,), jnp.int32),
            pltpu.VMEM((NBUF, GCHUNK, OUT_CH), jnp.float32),
            pltpu.SemaphoreType.DMA,
        ],
    )


def _gather_body(idx_hbm, table_hbm, out_hbm, idx_v, rows_v, gsem):
    wid = lax.axis_index("s") * NC + lax.axis_index("c")
    base = wid * (CHUNKS_PER_W * GCHUNK)
    pltpu.sync_copy(idx_hbm.at[pl.ds(base, CHUNKS_PER_W * GCHUNK)], idx_v)
    gathers = []
    for c in range(CHUNKS_PER_W):
        gathers.append(pltpu.async_copy(
            table_hbm.at[idx_v.at[pl.ds(c * GCHUNK, GCHUNK)]],
            rows_v.at[c % NBUF], gsem))
        if c >= NBUF - 1:
            _chunk_store(wid, c - NBUF + 1, rows_v, gathers[c - NBUF + 1],
                         out_hbm)
    for c in range(CHUNKS_PER_W - NBUF + 1, CHUNKS_PER_W):
        _chunk_store(wid, c, rows_v, gathers[c], out_hbm)
    @pl.when(wid == NW - 1)
    def _():
        pltpu.sync_copy(idx_hbm.at[pl.ds(NW * CHUNKS_PER_W * GCHUNK, GCHUNK)],
                        idx_v.at[pl.ds(0, GCHUNK)])
        pltpu.async_copy(
            table_hbm.at[idx_v.at[pl.ds(0, GCHUNK)]], rows_v.at[0],
            gsem).wait()
        pltpu.sync_copy(rows_v.at[0, pl.ds(0, 1)],
                        out_hbm.at[pl.ds(OUT_ROWS - 1, 1)])


def _chunk_store(wid, c, rows_v, gather, out_hbm):
    cid = wid * CHUNKS_PER_W + c
    off = cid * GCHUNK + jnp.where(cid < PRE_CHUNKS, 0, PROJ_ROWS - 2)
    off = pl.multiple_of(off, GCHUNK)
    gather.wait()
    pltpu.sync_copy(rows_v.at[c % NBUF], out_hbm.at[pl.ds(off, GCHUNK)])


def _finish_body(proj_hbm, out0_hbm, out_hbm, buf, buf2, sem, sem2):
    cp_in = pltpu.make_async_copy(proj_hbm, buf, sem)
    cp_in.start()
    cp_bnd = pltpu.make_async_copy(
        out0_hbm.at[pl.ds(PLACEHOLDER_POS + N_IMG, 8), :], buf2, sem2)
    cp_bnd.start()
    cp_in.wait()
    cp_bnd.wait()
    buf2[0:2, :] = buf[N_IMG:PROJ_ROWS, :]
    cp_out = pltpu.make_async_copy(
        buf.at[pl.ds(0, N_IMG)],
        out_hbm.at[pl.ds(PLACEHOLDER_POS, N_IMG), :], sem)
    cp_out.start()
    cp_out2 = pltpu.make_async_copy(
        buf2, out_hbm.at[pl.ds(PLACEHOLDER_POS + N_IMG, 8), :], sem2)
    cp_out2.start()
    cp_out.wait()
    cp_out2.wait()


def _finish(proj_ext, out0):
    return pl.pallas_call(
        _finish_body,
        in_specs=[pl.BlockSpec(memory_space=pl.ANY),
                  pl.BlockSpec(memory_space=pl.ANY)],
        out_specs=pl.BlockSpec(memory_space=pl.ANY),
        out_shape=jax.ShapeDtypeStruct((OUT_ROWS, OUT_CH), jnp.float32),
        scratch_shapes=[pltpu.VMEM((PROJ_ROWS, OUT_CH), jnp.float32),
                        pltpu.VMEM((8, OUT_CH), jnp.float32),
                        pltpu.SemaphoreType.DMA,
                        pltpu.SemaphoreType.DMA],
        input_output_aliases={1: 0},
    )(proj_ext, out0)


def kernel(instruction, img_token, embed_table, W1, b1, W2, b2,
           visual_start_token, visual_end_token):
    zeros2 = jnp.zeros((2,), jnp.int32)
    zpad = jnp.zeros((IDX_LEN - SEQ_LEN - 1,), jnp.int32)
    idx_sc = jnp.concatenate([
        instruction[:PLACEHOLDER_POS], zeros2,
        instruction[PLACEHOLDER_POS + 1:SEQ_LEN - 1],
        instruction[SEQ_LEN - 1:], zpad,
    ])
    out0 = _make_gather()(idx_sc, embed_table)
    proj_ext = _mlp(img_token.reshape(N_IMG, IN_CH), W1, b1, W2, b2,
                    visual_start_token.reshape(1, OUT_CH),
                    visual_end_token.reshape(1, OUT_CH))
    return _finish(proj_ext, out0)[None]
